# Initial kernel scaffold; baseline (speedup 1.0000x reference)
import functools

import jax
import jax.numpy as jnp
from jax import lax
from jax.experimental import pallas as pl
from jax.experimental.pallas import tpu as pltpu
from jax.experimental.pallas import tpu_sc as plsc

NC = 2
NS = 16
NW = NC * NS
K = 80

_MESH = plsc.VectorSubcoreMesh(core_axis_name="c", subcore_axis_name="s")


def _make_deg_kernel(n_nodes, n_chunks):
  rpt = n_nodes // NS

  @functools.partial(
      pl.kernel,
      out_type=jax.ShapeDtypeStruct((NC, n_nodes, 16), jnp.float32),
      mesh=_MESH,
      scratch_types=[
          pltpu.VMEM((n_chunks + 2, K), jnp.int32),
          pltpu.VMEM((K, 16), jnp.float32),
          pltpu.VMEM((rpt, 16), jnp.float32),
          pltpu.VMEM_SHARED((n_nodes, 16), jnp.float32),
      ],
  )
  def deg_kernel(dst_hbm, out_hbm, dst_all, ones_v, stage, acc):
    c = lax.axis_index("c")
    s = lax.axis_index("s")
    t = c * NS + s

    def fill_ones(j, _):
      ones_v[j, :] = jnp.full((16,), 1.0, jnp.float32)
      return 0
    lax.fori_loop(0, K, fill_ones, 0)

    def zero_row(j, _):
      stage[j, :] = jnp.zeros((16,), jnp.float32)
      return 0
    lax.fori_loop(0, rpt, zero_row, 0)
    pltpu.sync_copy(stage, acc.at[pl.ds(s * rpt, rpt)])
    pltpu.sync_copy(dst_hbm.at[t], dst_all)
    plsc.subcore_barrier()

    def body(i, _):
      pltpu.sync_copy(ones_v, acc.at[dst_all.at[i]], add=True)
      return 0
    lax.fori_loop(0, n_chunks, body, 0)
    plsc.subcore_barrier()

    pltpu.sync_copy(acc.at[pl.ds(s * rpt, rpt)], stage)
    pltpu.sync_copy(stage, out_hbm.at[c, pl.ds(s * rpt, rpt)])

  return deg_kernel


def _make_agg_kernel(n_nodes, n_chunks, feat):
  rpt = n_nodes // NS

  @functools.partial(
      pl.kernel,
      out_type=jax.ShapeDtypeStruct((NC, n_nodes, feat), jnp.float32),
      mesh=_MESH,
      scratch_types=[
          pltpu.VMEM((n_chunks + 2, K), jnp.int32),
          pltpu.VMEM((n_chunks + 2, K), jnp.int32),
          pltpu.VMEM((K, feat), jnp.float32),
          pltpu.VMEM((rpt, feat), jnp.float32),
          pltpu.VMEM_SHARED((n_nodes, feat), jnp.float32),
          pltpu.SemaphoreType.DMA,
      ],
  )
  def agg_kernel(u_hbm, src_hbm, dst_hbm, out_hbm,
                 src_all, dst_all, rows, stage, acc, sem):
    c = lax.axis_index("c")
    s = lax.axis_index("s")
    t = c * NS + s

    def zero_row(j, _):
      def zero_lane(f, _):
        stage[j, pl.ds(f * 16, 16)] = jnp.zeros((16,), jnp.float32)
        return 0
      lax.fori_loop(0, feat // 16, zero_lane, 0)
      return 0
    lax.fori_loop(0, rpt, zero_row, 0)
    pltpu.sync_copy(stage, acc.at[pl.ds(s * rpt, rpt)])
    pltpu.sync_copy(src_hbm.at[t], src_all)
    pltpu.sync_copy(dst_hbm.at[t], dst_all)
    plsc.subcore_barrier()

    def body(i, _):
      pltpu.async_copy(u_hbm.at[src_all.at[i]], rows, sem).wait()
      pltpu.sync_copy(rows, acc.at[dst_all.at[i]], add=True)
      return 0
    lax.fori_loop(0, n_chunks, body, 0)
    plsc.subcore_barrier()

    pltpu.sync_copy(acc.at[pl.ds(s * rpt, rpt)], stage)
    pltpu.sync_copy(stage, out_hbm.at[c, pl.ds(s * rpt, rpt)])

  return agg_kernel


def _tc_call(body, n, blk, in_specs_minor, out_minor, n_outs=1):
  grid = n // blk
  in_specs = []
  for m in in_specs_minor:
    if isinstance(m, tuple):
      in_specs.append(
          pl.BlockSpec(m, functools.partial(lambda r, i: (0,) * r, len(m))))
    else:
      in_specs.append(pl.BlockSpec((blk, m), lambda i: (i, 0)))
  if n_outs == 1:
    out_specs = pl.BlockSpec((blk, out_minor[0]), lambda i: (i, 0))
    out_shape = jax.ShapeDtypeStruct((n, out_minor[0]), jnp.float32)
  else:
    out_specs = [pl.BlockSpec((blk, m), lambda i: (i, 0)) for m in out_minor]
    out_shape = [jax.ShapeDtypeStruct((n, m), jnp.float32) for m in out_minor]
  return pl.pallas_call(
      body, grid=(grid,), in_specs=in_specs, out_specs=out_specs,
      out_shape=out_shape)


def _t0_body(x_ref, d0_ref, d1_ref, u_ref, dinv_ref):
  deg = d0_ref[:, 0:1] + d1_ref[:, 0:1] + 1.0
  dinv = lax.rsqrt(deg)
  dinv_ref[...] = jnp.broadcast_to(dinv, dinv_ref.shape)
  u_ref[...] = x_ref[...] * dinv


def _t1_body(p0_ref, p1_ref, u_ref, dinv_ref, w1_ref, b1_ref, w2_ref, g1_ref):
  dinv = dinv_ref[...]
  z = dinv * (p0_ref[...] + p1_ref[...] + u_ref[...])
  h = jnp.maximum(
      jnp.dot(z, w1_ref[...], preferred_element_type=jnp.float32)
      + b1_ref[...], 0.0)
  g1_ref[...] = dinv * jnp.dot(h, w2_ref[...],
                               preferred_element_type=jnp.float32)


def _t2_body(q0_ref, q1_ref, g1_ref, dinv_ref, w3_ref, b2_ref, u3_ref):
  dinv = dinv_ref[...]
  h2 = jnp.maximum(dinv * (q0_ref[...] + q1_ref[...] + g1_ref[...])
                   + b2_ref[...], 0.0)
  u3_ref[...] = dinv[:, :u3_ref.shape[1]] * jnp.dot(
      h2, w3_ref[...], preferred_element_type=jnp.float32)


def _t3_body(r0_ref, r1_ref, u3_ref, dinv_ref, b3_ref, o_ref, *, n_cls):
  fp = u3_ref.shape[1]
  z = dinv_ref[:, :fp] * (r0_ref[...] + r1_ref[...] + u3_ref[...]) + b3_ref[...]
  zc = z[:, :n_cls]
  m = jnp.max(zc, axis=1, keepdims=True)
  lse = jnp.log(jnp.sum(jnp.exp(zc - m), axis=1, keepdims=True))
  o_ref[...] = z - m - lse


def kernel(x, edge_index, W1, b1, W2, b2, W3, b3):
  n, d_in = x.shape
  e = edge_index.shape[1]
  h1 = W1.shape[1]
  h2 = W2.shape[1]
  c_cls = W3.shape[1]
  fp = 48
  epw = e // NW
  n_chunks = epw // K
  blk = 1000

  src = edge_index[0].astype(jnp.int32)
  dst = edge_index[1].astype(jnp.int32)
  pad = (n_chunks + 2) * K - epw
  src3 = jnp.pad(src.reshape(NW, epw), ((0, 0), (0, pad))).reshape(
      NW, n_chunks + 2, K)
  dst3 = jnp.pad(dst.reshape(NW, epw), ((0, 0), (0, pad))).reshape(
      NW, n_chunks + 2, K)
  w3p = jnp.pad(W3, ((0, 0), (0, fp - c_cls)))
  b1r = b1.reshape(1, h1)
  b2r = b2.reshape(1, h2)
  b3r = jnp.pad(b3, (0, fp - c_cls)).reshape(1, fp)

  deg_parts = _make_deg_kernel(n, n_chunks)(dst3)
  agg128 = _make_agg_kernel(n, n_chunks, d_in)
  agg48 = _make_agg_kernel(n, n_chunks, fp)

  u, dinvb = _tc_call(_t0_body, n, blk, [d_in, 16, 16], [d_in, d_in],
                      n_outs=2)(x, deg_parts[0], deg_parts[1])
  p = agg128(u, src3, dst3)
  g1 = _tc_call(_t1_body, n, blk,
                [d_in, d_in, d_in, d_in, (d_in, h1), (1, h1), (h1, h2)],
                [h2])(p[0], p[1], u, dinvb, W1, b1r, W2)
  q = agg128(g1, src3, dst3)
  u3 = _tc_call(_t2_body, n, blk,
                [h2, h2, h2, d_in, (h2, fp), (1, h2)],
                [fp])(q[0], q[1], g1, dinvb, w3p, b2r)
  r = agg48(u3, src3, dst3)
  o = _tc_call(functools.partial(_t3_body, n_cls=c_cls), n, blk,
               [fp, fp, fp, d_in, (1, fp)],
               [fp])(r[0], r[1], u3, dinvb, b3r)
  return o[:, :c_cls]


# speedup vs baseline: 13.0271x; 13.0271x over previous
import functools

import jax
import jax.numpy as jnp
from jax import lax
from jax.experimental import pallas as pl
from jax.experimental.pallas import tpu as pltpu
from jax.experimental.pallas import tpu_sc as plsc

NC = 2
NS = 16
NW = NC * NS
K = 80

_MESH = plsc.VectorSubcoreMesh(core_axis_name="c", subcore_axis_name="s")


def _make_agg_kernel(n_pad, epw, gather):
  rpt = n_pad // NS
  n_chunks = epw // K
  n_wo = rpt // K
  feat = 128

  scratch = [
      pltpu.VMEM((K,), jnp.int32),
      pltpu.VMEM((K, feat), jnp.float32),
      pltpu.VMEM_SHARED((n_pad, feat), jnp.float32),
      pltpu.SemaphoreType.DMA,
  ]
  if gather:
    scratch.insert(0, pltpu.VMEM((K,), jnp.int32))

  def body(*args):
    if gather:
      u_hbm, src_hbm, dst_hbm, out_hbm, si, di, rows, acc, sem = args
    else:
      dst_hbm, out_hbm, di, rows, acc, sem = args
    c = lax.axis_index("c")
    s = lax.axis_index("s")
    t = c * NS + s
    fill = jnp.zeros((16,), jnp.float32) if gather else jnp.full(
        (16,), 1.0, jnp.float32)

    def fill_row(j, _):
      def fill_lane(f, _):
        rows[j, pl.ds(f * 16, 16)] = fill
        return 0
      lax.fori_loop(0, feat // 16, fill_lane, 0)
      return 0
    lax.fori_loop(0, K, fill_row, 0)

    if gather:
      def zero_acc(j, _):
        pltpu.sync_copy(rows, acc.at[pl.ds(s * rpt + j * K, K)])
        return 0
      lax.fori_loop(0, n_wo, zero_acc, 0)
    else:
      def zero_lane(f, _):
        rows[0, pl.ds(f * 16, 16)] = jnp.zeros((16,), jnp.float32)
        return 0
      lax.fori_loop(0, feat // 16, zero_lane, 0)

      def zero_acc(j, _):
        pltpu.sync_copy(rows.at[pl.ds(0, 1)], acc.at[pl.ds(s * rpt + j, 1)])
        return 0
      lax.fori_loop(0, rpt, zero_acc, 0)

      def refill_lane(f, _):
        rows[0, pl.ds(f * 16, 16)] = jnp.full((16,), 1.0, jnp.float32)
        return 0
      lax.fori_loop(0, feat // 16, refill_lane, 0)
    plsc.subcore_barrier()

    def loop(i, _):
      base = t * epw + i * K
      if gather:
        pltpu.sync_copy(src_hbm.at[pl.ds(base, K)], si)
      pltpu.sync_copy(dst_hbm.at[pl.ds(base, K)], di)
      if gather:
        pltpu.async_copy(u_hbm.at[si], rows, sem).wait()
      pltpu.sync_copy(rows, acc.at[di], add=True)
      return 0
    lax.fori_loop(0, n_chunks, loop, 0)
    plsc.subcore_barrier()

    def writeout(j, _):
      pltpu.sync_copy(acc.at[pl.ds(s * rpt + j * K, K)], rows)
      pltpu.sync_copy(rows, out_hbm.at[c, pl.ds(s * rpt + j * K, K)])
      return 0
    lax.fori_loop(0, n_wo, writeout, 0)

  return pl.kernel(
      body,
      out_type=jax.ShapeDtypeStruct((NC, n_pad, feat), jnp.float32),
      mesh=_MESH,
      scratch_types=scratch,
  )


def _tc_call(body, n, blk, in_specs_minor, out_minor, n_outs=1):
  grid = n // blk
  in_specs = []
  for m in in_specs_minor:
    if isinstance(m, tuple):
      in_specs.append(
          pl.BlockSpec(m, functools.partial(lambda r, i: (0,) * r, len(m))))
    else:
      in_specs.append(pl.BlockSpec((blk, m), lambda i: (i, 0)))
  if n_outs == 1:
    out_specs = pl.BlockSpec((blk, out_minor[0]), lambda i: (i, 0))
    out_shape = jax.ShapeDtypeStruct((n, out_minor[0]), jnp.float32)
  else:
    out_specs = [pl.BlockSpec((blk, m), lambda i: (i, 0)) for m in out_minor]
    out_shape = [jax.ShapeDtypeStruct((n, m), jnp.float32) for m in out_minor]
  return pl.pallas_call(
      body, grid=(grid,), in_specs=in_specs, out_specs=out_specs,
      out_shape=out_shape)


def _t0_body(x_ref, d0_ref, d1_ref, u_ref, dinv_ref):
  deg = d0_ref[:, 0:1] + d1_ref[:, 0:1] + 1.0
  dinv = lax.rsqrt(deg)
  dinv_ref[...] = jnp.broadcast_to(dinv, dinv_ref.shape)
  u_ref[...] = x_ref[...] * dinv


def _t1_body(p0_ref, p1_ref, u_ref, dinv_ref, w1_ref, b1_ref, w2_ref, g1_ref):
  dinv = dinv_ref[...]
  z = dinv * (p0_ref[...] + p1_ref[...] + u_ref[...])
  h = jnp.maximum(
      jnp.dot(z, w1_ref[...], preferred_element_type=jnp.float32)
      + b1_ref[...], 0.0)
  g1_ref[...] = dinv * jnp.dot(h, w2_ref[...],
                               preferred_element_type=jnp.float32)


def _t2_body(q0_ref, q1_ref, g1_ref, dinv_ref, b2_ref, u3_ref):
  dinv = dinv_ref[...]
  h2 = jnp.maximum(dinv * (q0_ref[...] + q1_ref[...] + g1_ref[...])
                   + b2_ref[...], 0.0)
  u3_ref[...] = dinv * h2


def _t3_body(r0_ref, r1_ref, u3_ref, dinv_ref, w3_ref, b3_ref, o_ref, *, n_cls):
  s3 = dinv_ref[...] * (r0_ref[...] + r1_ref[...] + u3_ref[...])
  z = jnp.dot(s3, w3_ref[...], preferred_element_type=jnp.float32) + b3_ref[...]
  zc = z[:, :n_cls]
  m = jnp.max(zc, axis=1, keepdims=True)
  lse = jnp.log(jnp.sum(jnp.exp(zc - m), axis=1, keepdims=True))
  o_ref[...] = z - m - lse


def kernel(x, edge_index, W1, b1, W2, b2, W3, b3):
  n, d_in = x.shape
  e = edge_index.shape[1]
  h1 = W1.shape[1]
  h2 = W2.shape[1]
  c_cls = W3.shape[1]
  fp = 48
  epw = e // NW
  blk = 1000
  n_pad = -(-n // (NS * K)) * (NS * K)

  src = edge_index[0].astype(jnp.int32)
  dst = edge_index[1].astype(jnp.int32)
  w3p = jnp.pad(W3, ((0, 0), (0, fp - c_cls)))
  b1r = b1.reshape(1, h1)
  b2r = b2.reshape(1, h2)
  b3r = jnp.pad(b3, (0, fp - c_cls)).reshape(1, fp)

  deg_parts = _make_agg_kernel(n_pad, epw, gather=False)(dst)
  agg128 = _make_agg_kernel(n_pad, epw, gather=True)

  u, dinvb = _tc_call(_t0_body, n, blk, [d_in, 128, 128], [d_in, d_in],
                      n_outs=2)(x, deg_parts[0], deg_parts[1])
  p = agg128(u, src, dst)
  g1 = _tc_call(_t1_body, n, blk,
                [d_in, d_in, d_in, d_in, (d_in, h1), (1, h1), (h1, h2)],
                [h2])(p[0], p[1], u, dinvb, W1, b1r, W2)
  q = agg128(g1, src, dst)
  u3 = _tc_call(_t2_body, n, blk,
                [h2, h2, h2, d_in, (1, h2)],
                [d_in])(q[0], q[1], g1, dinvb, b2r)
  r = agg128(u3, src, dst)
  o = _tc_call(functools.partial(_t3_body, n_cls=c_cls), n, blk,
               [d_in, d_in, d_in, d_in, (h2, fp), (1, fp)],
               [fp])(r[0], r[1], u3, dinvb, w3p, b3r)
  return o[:, :c_cls]


# speedup vs baseline: 18.1182x; 1.3908x over previous
import functools

import jax
import jax.numpy as jnp
from jax import lax
from jax.experimental import pallas as pl
from jax.experimental.pallas import tpu as pltpu
from jax.experimental.pallas import tpu_sc as plsc

NC = 2
NS = 16
NW = NC * NS
CK = 96

_MESH = plsc.VectorSubcoreMesh(core_axis_name="c", subcore_axis_name="s")


def _make_agg_kernel(n_pad, epw_pad, gather):
  rpt = n_pad // NS
  n_ch = epw_pad // CK
  feat = 128
  WO = 80
  n_wo = rpt // WO

  scratch = []
  if gather:
    scratch += [pltpu.VMEM((CK,), jnp.int32) for _ in range(3)]
  scratch += [pltpu.VMEM((CK,), jnp.int32) for _ in range(3)]
  n_rows = 3 if gather else 1
  scratch += [pltpu.VMEM((CK, feat), jnp.float32) for _ in range(n_rows)]
  scratch += [pltpu.VMEM_SHARED((n_pad, feat), jnp.float32)]
  n_sem = 6 if gather else 3
  scratch += [pltpu.SemaphoreType.DMA for _ in range(n_sem)]

  def body(*args):
    if gather:
      (u_hbm, src_hbm, dst_hbm, out_hbm,
       si0, si1, si2, di0, di1, di2, r0, r1, r2, acc,
       g0, g1, g2, s0, s1, s2) = args
      si = (si0, si1, si2)
      rows = (r0, r1, r2)
      gsem = (g0, g1, g2)
      ssem = (s0, s1, s2)
    else:
      dst_hbm, out_hbm, di0, di1, di2, r0, acc, s0, s1, s2 = args
      rows = (r0, r0, r0)
      ssem = (s0, s1, s2)
    di = (di0, di1, di2)
    c = lax.axis_index("c")
    s = lax.axis_index("s")
    t = c * NS + s
    ebase = t * epw_pad

    def fill_row(j, _):
      def fill_lane(f, _):
        r0[j, pl.ds(f * 16, 16)] = jnp.zeros((16,), jnp.float32)
        return 0
      lax.fori_loop(0, feat // 16, fill_lane, 0)
      return 0
    lax.fori_loop(0, CK, fill_row, 0)

    def zero_acc(j, _):
      pltpu.sync_copy(r0.at[pl.ds(0, WO)], acc.at[pl.ds(s * rpt + j * WO, WO)])
      return 0
    lax.fori_loop(0, n_wo, zero_acc, 0)

    if not gather:
      def ones_row(j, _):
        def ones_lane(f, _):
          r0[j, pl.ds(f * 16, 16)] = jnp.full((16,), 1.0, jnp.float32)
          return 0
        lax.fori_loop(0, feat // 16, ones_lane, 0)
        return 0
      lax.fori_loop(0, CK, ones_row, 0)
    plsc.subcore_barrier()

    def prep(i, b):
      pltpu.sync_copy(dst_hbm.at[pl.ds(ebase + i * CK, CK)], di[b])
      if gather:
        pltpu.sync_copy(src_hbm.at[pl.ds(ebase + i * CK, CK)], si[b])
        pltpu.async_copy(u_hbm.at[si[b]], rows[b], gsem[b])

    prep(0, 0)
    prep(1, 1)

    def group(j, _):
      for k in range(3):
        i = 3 * j + k
        b = k
        b2 = (k + 2) % 3
        if gather:
          pltpu.make_async_copy(u_hbm.at[si[b]], rows[b], gsem[b]).wait()
        pltpu.async_copy(rows[b], acc.at[di[b]], ssem[b], add=True)

        @pl.when(jnp.logical_and(i >= 1, i + 2 < n_ch))
        def _():
          pltpu.make_async_copy(rows[b2], acc.at[di[b2]], ssem[b2]).wait()

        @pl.when(i + 2 < n_ch)
        def _():
          prep(i + 2, b2)
      return 0
    lax.fori_loop(0, n_ch // 3, group, 0)

    for b in range(3):
      pltpu.make_async_copy(rows[b], acc.at[di[b]], ssem[b]).wait()
    plsc.subcore_barrier()

    def writeout(j, _):
      pltpu.sync_copy(acc.at[pl.ds(s * rpt + j * WO, WO)], r0.at[pl.ds(0, WO)])
      pltpu.sync_copy(r0.at[pl.ds(0, WO)],
                      out_hbm.at[c, pl.ds(s * rpt + j * WO, WO)])
      return 0
    lax.fori_loop(0, n_wo, writeout, 0)

  return pl.kernel(
      body,
      out_type=jax.ShapeDtypeStruct((NC, n_pad, feat), jnp.float32),
      mesh=_MESH,
      scratch_types=scratch,
  )


def _tc_call(body, n, blk, in_specs_minor, out_minor, n_outs=1):
  grid = n // blk
  in_specs = []
  for m in in_specs_minor:
    if isinstance(m, tuple):
      in_specs.append(
          pl.BlockSpec(m, functools.partial(lambda r, i: (0,) * r, len(m))))
    else:
      in_specs.append(pl.BlockSpec((blk, m), lambda i: (i, 0)))
  if n_outs == 1:
    out_specs = pl.BlockSpec((blk, out_minor[0]), lambda i: (i, 0))
    out_shape = jax.ShapeDtypeStruct((n, out_minor[0]), jnp.float32)
  else:
    out_specs = [pl.BlockSpec((blk, m), lambda i: (i, 0)) for m in out_minor]
    out_shape = [jax.ShapeDtypeStruct((n, m), jnp.float32) for m in out_minor]
  return pl.pallas_call(
      body, grid=(grid,), in_specs=in_specs, out_specs=out_specs,
      out_shape=out_shape)


def _t0_body(x_ref, d0_ref, d1_ref, u_ref, dinv_ref):
  deg = d0_ref[:, 0:1] + d1_ref[:, 0:1] + 1.0
  dinv = lax.rsqrt(deg)
  dinv_ref[...] = jnp.broadcast_to(dinv, dinv_ref.shape)
  u_ref[...] = x_ref[...] * dinv


def _t1_body(p0_ref, p1_ref, u_ref, dinv_ref, w1_ref, b1_ref, w2_ref, g1_ref):
  dinv = dinv_ref[...]
  z = dinv * (p0_ref[...] + p1_ref[...] + u_ref[...])
  h = jnp.maximum(
      jnp.dot(z, w1_ref[...], preferred_element_type=jnp.float32)
      + b1_ref[...], 0.0)
  g1_ref[...] = dinv * jnp.dot(h, w2_ref[...],
                               preferred_element_type=jnp.float32)


def _t2_body(q0_ref, q1_ref, g1_ref, dinv_ref, b2_ref, u3_ref):
  dinv = dinv_ref[...]
  h2 = jnp.maximum(dinv * (q0_ref[...] + q1_ref[...] + g1_ref[...])
                   + b2_ref[...], 0.0)
  u3_ref[...] = dinv * h2


def _t3_body(r0_ref, r1_ref, u3_ref, dinv_ref, w3_ref, b3_ref, o_ref, *, n_cls):
  s3 = dinv_ref[...] * (r0_ref[...] + r1_ref[...] + u3_ref[...])
  z = jnp.dot(s3, w3_ref[...], preferred_element_type=jnp.float32) + b3_ref[...]
  zc = z[:, :n_cls]
  m = jnp.max(zc, axis=1, keepdims=True)
  lse = jnp.log(jnp.sum(jnp.exp(zc - m), axis=1, keepdims=True))
  o_ref[...] = z - m - lse


def kernel(x, edge_index, W1, b1, W2, b2, W3, b3):
  n, d_in = x.shape
  e = edge_index.shape[1]
  h1 = W1.shape[1]
  h2 = W2.shape[1]
  c_cls = W3.shape[1]
  fp = 48
  epw = e // NW
  blk = 1000
  n_pad = -(-n // (80 * NS)) * (80 * NS)
  n_ch = -(-epw // CK)
  epw_pad = n_ch * CK

  epad = ((0, 0), (0, epw_pad - epw))
  src = jnp.pad(edge_index[0].astype(jnp.int32).reshape(NW, epw),
                epad).reshape(-1)
  dst = jnp.pad(edge_index[1].astype(jnp.int32).reshape(NW, epw),
                epad, constant_values=n).reshape(-1)
  w3p = jnp.pad(W3, ((0, 0), (0, fp - c_cls)))
  b1r = b1.reshape(1, h1)
  b2r = b2.reshape(1, h2)
  b3r = jnp.pad(b3, (0, fp - c_cls)).reshape(1, fp)

  deg_parts = _make_agg_kernel(n_pad, epw_pad, gather=False)(dst)
  agg128 = _make_agg_kernel(n_pad, epw_pad, gather=True)

  u, dinvb = _tc_call(_t0_body, n, blk, [d_in, 128, 128], [d_in, d_in],
                      n_outs=2)(x, deg_parts[0], deg_parts[1])
  p = agg128(u, src, dst)
  g1 = _tc_call(_t1_body, n, blk,
                [d_in, d_in, d_in, d_in, (d_in, h1), (1, h1), (h1, h2)],
                [h2])(p[0], p[1], u, dinvb, W1, b1r, W2)
  q = agg128(g1, src, dst)
  u3 = _tc_call(_t2_body, n, blk,
                [h2, h2, h2, d_in, (1, h2)],
                [d_in])(q[0], q[1], g1, dinvb, b2r)
  r = agg128(u3, src, dst)
  o = _tc_call(functools.partial(_t3_body, n_cls=c_cls), n, blk,
               [d_in, d_in, d_in, d_in, (h2, fp), (1, fp)],
               [fp])(r[0], r[1], u3, dinvb, w3p, b3r)
  return o[:, :c_cls]
